# baseline (speedup 1.0000x reference)
import functools

import jax
import jax.numpy as jnp
from jax import lax
from jax.experimental import pallas as pl
from jax.experimental.pallas import tpu as pltpu
from jax.experimental.pallas import tpu_sc as plsc

N = 10000
E = 320000
C = 128
K = 27
EPS = 1e-5

CH = 64
NWORKERS = 32
NJ = 160
NB = 4
LAG = 2
SG = 16
NSG = NJ // SG
E_PAD = NWORKERS * NJ * CH
NPAD = 10240
ROWS_PER_TILE = NPAD // 16


def _mm_body(x_ref, w_ref, o_ref):
    x = x_ref[...]
    for k in range(K):
        o_ref[k] = jnp.dot(x, w_ref[k], preferred_element_type=jnp.float32)


def _transform(feats, W_conv):
    BN = 400
    return pl.pallas_call(
        _mm_body,
        grid=(N // BN,),
        in_specs=[
            pl.BlockSpec((BN, C), lambda i: (i, 0)),
            pl.BlockSpec((K, C, C), lambda i: (0, 0, 0)),
        ],
        out_specs=pl.BlockSpec((K, BN, C), lambda i: (0, i, 0)),
        out_shape=jax.ShapeDtypeStruct((K, N, C), jnp.float32),
    )(feats, W_conv)


_MESH = plsc.VectorSubcoreMesh(core_axis_name="c", subcore_axis_name="s")


@functools.partial(
    pl.kernel,
    out_type=jax.ShapeDtypeStruct((2, NPAD, C), jnp.float32),
    mesh=_MESH,
    scratch_types=[
        pltpu.VMEM((2, SG, CH), jnp.int32),
        pltpu.VMEM((2, SG, CH), jnp.int32),
        pltpu.VMEM((NB, CH, C), jnp.float32),
        pltpu.VMEM_SHARED((NPAD, C), jnp.float32),
        pltpu.SemaphoreType.DMA,
        pltpu.SemaphoreType.DMA,
        pltpu.SemaphoreType.DMA,
        pltpu.SemaphoreType.DMA,
        pltpu.SemaphoreType.DMA,
        pltpu.SemaphoreType.DMA,
        pltpu.SemaphoreType.DMA,
        pltpu.SemaphoreType.DMA,
        pltpu.SemaphoreType.DMA,
        pltpu.SemaphoreType.DMA,
    ],
)
def _sc_scatter(idx_hbm, t_hbm, zeros_hbm, out_hbm,
                rid_v, dst_v, rows_v, acc_sh,
                sem_rid, sem_dst, g0, g1, g2, g3, s0, s1, s2, s3):
    gsem = (g0, g1, g2, g3)
    ssem = (s0, s1, s2, s3)
    cid = lax.axis_index("c")
    sid = lax.axis_index("s")
    wid = sid * 2 + cid
    base = wid * NJ

    def stage_fire(sg, p):
        sl = pl.ds(base + sg * SG, SG)
        pltpu.async_copy(idx_hbm.at[0, sl], rid_v.at[p], sem_rid)
        pltpu.async_copy(idx_hbm.at[1, sl], dst_v.at[p], sem_dst)

    def stage_wait(sg, p):
        sl = pl.ds(base + sg * SG, SG)
        pltpu.make_async_copy(idx_hbm.at[0, sl], rid_v.at[p], sem_rid).wait()
        pltpu.make_async_copy(idx_hbm.at[1, sl], dst_v.at[p], sem_dst).wait()

    def gather_fire(p, c):
        pltpu.async_copy(t_hbm.at[rid_v.at[p, c]], rows_v.at[c % NB],
                         gsem[c % NB])

    def gather_wait(p, c):
        pltpu.make_async_copy(t_hbm.at[rid_v.at[p, c]], rows_v.at[c % NB],
                              gsem[c % NB]).wait()

    def scatter_fire(p, c):
        pltpu.async_copy(rows_v.at[c % NB], acc_sh.at[dst_v.at[p, c]],
                         ssem[c % NB], add=True)

    def scatter_wait(p, c):
        pltpu.make_async_copy(rows_v.at[c % NB], acc_sh.at[dst_v.at[p, c]],
                              ssem[c % NB]).wait()

    stage_fire(0, 0)
    pltpu.sync_copy(zeros_hbm,
                    acc_sh.at[pl.ds(sid * ROWS_PER_TILE, ROWS_PER_TILE)])
    plsc.subcore_barrier()

    def double_group(g, carry):
        for p in (0, 1):
            sg = g * 2 + p
            for c in range(SG):
                if c == 0:
                    stage_wait(sg, p)
                if c >= NB:
                    scatter_wait(p, c - NB)
                elif p == 1:
                    scatter_wait(0, SG - NB + c)
                else:
                    @pl.when(g > 0)
                    def _():
                        scatter_wait(1, SG - NB + c)
                gather_fire(p, c)
                if c >= LAG:
                    gather_wait(p, c - LAG)
                    scatter_fire(p, c - LAG)
                elif p == 1:
                    gather_wait(0, SG - LAG + c)
                    scatter_fire(0, SG - LAG + c)
                else:
                    @pl.when(g > 0)
                    def _():
                        gather_wait(1, SG - LAG + c)
                        scatter_fire(1, SG - LAG + c)
                if c == NB:
                    if p == 0:
                        stage_fire(sg + 1, 1)
                    else:
                        @pl.when(g < NSG // 2 - 1)
                        def _():
                            stage_fire(sg + 1, 0)
        return carry

    lax.fori_loop(0, NSG // 2, double_group, 0)
    for c in range(SG - LAG, SG):
        gather_wait(1, c)
        scatter_fire(1, c)
    for c in range(SG - NB, SG):
        scatter_wait(1, c)
    plsc.subcore_barrier()

    pltpu.sync_copy(acc_sh.at[pl.ds(sid * ROWS_PER_TILE, ROWS_PER_TILE)],
                    out_hbm.at[cid, pl.ds(sid * ROWS_PER_TILE, ROWS_PER_TILE)])


def _epi_body(p_ref, cb_ref, wl_ref, bl_ref, g_ref, b_ref, o_ref):
    conv = p_ref[0] + p_ref[1] + cb_ref[...]
    lin = lax.dot_general(conv, wl_ref[...], (((1,), (1,)), ((), ())),
                          preferred_element_type=jnp.float32) + bl_ref[...]
    mean = jnp.mean(lin, axis=1, keepdims=True)
    cent = lin - mean
    var = jnp.mean(cent * cent, axis=1, keepdims=True)
    o_ref[...] = cent * lax.rsqrt(var + EPS) * g_ref[...] + b_ref[...]


def _epilogue(partials, conv_bias, W_lin, b_lin, ln_gamma, ln_beta):
    BN = 1000
    return pl.pallas_call(
        _epi_body,
        grid=(N // BN,),
        in_specs=[
            pl.BlockSpec((2, BN, C), lambda i: (0, i, 0)),
            pl.BlockSpec((1, C), lambda i: (0, 0)),
            pl.BlockSpec((C, C), lambda i: (0, 0)),
            pl.BlockSpec((1, C), lambda i: (0, 0)),
            pl.BlockSpec((1, C), lambda i: (0, 0)),
            pl.BlockSpec((1, C), lambda i: (0, 0)),
        ],
        out_specs=pl.BlockSpec((BN, C), lambda i: (i, 0)),
        out_shape=jax.ShapeDtypeStruct((N, C), jnp.float32),
    )(partials, conv_bias.reshape(1, C), W_lin, b_lin.reshape(1, C),
      ln_gamma.reshape(1, C), ln_beta.reshape(1, C))


def kernel(feats, edge_index, edge_kernel, W_conv, conv_bias, W_lin, b_lin,
           ln_gamma, ln_beta):
    T = _transform(feats, W_conv).reshape(K * N, C)
    zeros = jnp.zeros((ROWS_PER_TILE, C), dtype=jnp.float32)
    pad = E_PAD - E
    pad_iota = jnp.arange(pad, dtype=jnp.int32)
    addend = jnp.stack([edge_kernel * N, jnp.zeros((E,), jnp.int32)])
    pad_block = jnp.stack([pad_iota, N + (pad_iota & 127)])
    idx_all = jnp.concatenate([edge_index + addend, pad_block],
                              axis=1).reshape(2, E_PAD // CH, CH)
    partials = _sc_scatter(idx_all, T, zeros)
    return _epilogue(partials, conv_bias, W_lin, b_lin, ln_gamma, ln_beta)


# speedup vs baseline: 1.0264x; 1.0264x over previous
import functools

import jax
import jax.numpy as jnp
from jax import lax
from jax.experimental import pallas as pl
from jax.experimental.pallas import tpu as pltpu
from jax.experimental.pallas import tpu_sc as plsc

N = 10000
E = 320000
C = 128
K = 27
EPS = 1e-5

CH = 64
NWORKERS = 32
NCHUNK = E // CH
NJ = 156
NB = 4
NI = 8
LAG = 2
NPAD = 10240
ROWS_PER_TILE = NPAD // 16


def _mm_body(x_ref, w_ref, o_ref):
    x = x_ref[...]
    for k in range(K):
        o_ref[k] = jnp.dot(x, w_ref[k], preferred_element_type=jnp.float32)


def _transform(feats, W_conv):
    BN = 400
    return pl.pallas_call(
        _mm_body,
        grid=(N // BN,),
        in_specs=[
            pl.BlockSpec((BN, C), lambda i: (i, 0)),
            pl.BlockSpec((K, C, C), lambda i: (0, 0, 0)),
        ],
        out_specs=pl.BlockSpec((K, BN, C), lambda i: (0, i, 0)),
        out_shape=jax.ShapeDtypeStruct((K, N, C), jnp.float32),
    )(feats, W_conv)


_MESH = plsc.VectorSubcoreMesh(core_axis_name="c", subcore_axis_name="s")


@functools.partial(
    pl.kernel,
    out_type=jax.ShapeDtypeStruct((2, NPAD, C), jnp.float32),
    mesh=_MESH,
    scratch_types=(
        [
            pltpu.VMEM((NI, CH), jnp.int32),
            pltpu.VMEM((NI, CH), jnp.int32),
            pltpu.VMEM((NI, CH), jnp.int32),
            pltpu.VMEM((NB, CH, C), jnp.float32),
            pltpu.VMEM_SHARED((NPAD, C), jnp.float32),
        ]
        + [pltpu.SemaphoreType.DMA] * (NI + NB + NB)
    ),
)
def _sc_scatter(ei_hbm, ek_hbm, t_hbm, zeros_hbm, out_hbm,
                rid_v, kern_v, dst_v, rows_v, acc_sh, *sems):
    isem = sems[:NI]
    gsem = sems[NI:NI + NB]
    ssem = sems[NI + NB:]
    cid = lax.axis_index("c")
    sid = lax.axis_index("s")
    wid = sid * 2 + cid
    start = wid * NJ + jnp.minimum(wid, 8)
    has_extra = wid < 8

    def idx_fire(n, u):
        sl = pl.ds((start + n) * CH, CH)
        pltpu.async_copy(ei_hbm.at[0, sl], rid_v.at[u], isem[u])
        pltpu.async_copy(ek_hbm.at[sl], kern_v.at[u], isem[u])
        pltpu.async_copy(ei_hbm.at[1, sl], dst_v.at[u], isem[u])

    def idx_wait(n, u):
        sl = pl.ds((start + n) * CH, CH)
        pltpu.make_async_copy(ei_hbm.at[0, sl], rid_v.at[u], isem[u]).wait()
        pltpu.make_async_copy(ek_hbm.at[sl], kern_v.at[u], isem[u]).wait()
        pltpu.make_async_copy(ei_hbm.at[1, sl], dst_v.at[u], isem[u]).wait()

    def rid_compute(u):
        for i in range(CH // 16):
            sl = pl.ds(i * 16, 16)
            rid_v[u, sl] = kern_v[u, sl] * N + rid_v[u, sl]

    def gather_fire(u, q):
        pltpu.async_copy(t_hbm.at[rid_v.at[u]], rows_v.at[q], gsem[q])

    def gather_wait(u, q):
        pltpu.make_async_copy(t_hbm.at[rid_v.at[u]], rows_v.at[q],
                              gsem[q]).wait()

    def scatter_fire(u, q):
        pltpu.async_copy(rows_v.at[q], acc_sh.at[dst_v.at[u]], ssem[q],
                         add=True)

    def scatter_wait(u, q):
        pltpu.make_async_copy(rows_v.at[q], acc_sh.at[dst_v.at[u]],
                              ssem[q]).wait()

    for n in range(NB):
        idx_fire(n, n)
    pltpu.sync_copy(zeros_hbm,
                    acc_sh.at[pl.ds(sid * ROWS_PER_TILE, ROWS_PER_TILE)])
    plsc.subcore_barrier()

    def make_round(parity):
        def body(n0):
            for j in range(NB):
                n = n0 + j
                u = parity * NB + j
                u_next = (1 - parity) * NB + j
                q = j

                @pl.when(n >= NB)
                def _():
                    scatter_wait(u_next, q)

                @pl.when(n + NB < NJ)
                def _():
                    idx_fire(n + NB, u_next)
                idx_wait(n, u)
                rid_compute(u)
                jl = j - LAG
                ul = parity * NB + jl if jl >= 0 else (1 - parity) * NB + NB + jl
                ql = jl % NB

                @pl.when(n >= LAG)
                def _():
                    gather_wait(ul, ql)
                    scatter_fire(ul, ql)
                gather_fire(u, q)
        return body

    round_even = make_round(0)
    round_odd = make_round(1)

    def double_round(rr, carry):
        round_even(rr * 2 * NB)
        round_odd(rr * 2 * NB + NB)
        return carry

    lax.fori_loop(0, NJ // (2 * NB), double_round, 0)
    round_even(NJ - NB)
    for t in range(NJ - LAG, NJ):
        j = t - (NJ - NB)
        gather_wait(j, j)
        scatter_fire(j, j)
    for t in range(NJ - NB, NJ):
        j = t - (NJ - NB)
        scatter_wait(j, j)

    @pl.when(has_extra)
    def _():
        sl = pl.ds((start + NJ) * CH, CH)
        pltpu.sync_copy(ei_hbm.at[0, sl], rid_v.at[0])
        pltpu.sync_copy(ek_hbm.at[sl], kern_v.at[0])
        pltpu.sync_copy(ei_hbm.at[1, sl], dst_v.at[0])
        rid_compute(0)
        pltpu.async_copy(t_hbm.at[rid_v.at[0]], rows_v.at[0], gsem[0]).wait()
        pltpu.async_copy(rows_v.at[0], acc_sh.at[dst_v.at[0]], ssem[0],
                         add=True).wait()

    plsc.subcore_barrier()
    pltpu.sync_copy(acc_sh.at[pl.ds(sid * ROWS_PER_TILE, ROWS_PER_TILE)],
                    out_hbm.at[cid, pl.ds(sid * ROWS_PER_TILE, ROWS_PER_TILE)])


def _epi_body(p_ref, cb_ref, wl_ref, bl_ref, g_ref, b_ref, o_ref):
    conv = p_ref[0] + p_ref[1] + cb_ref[...]
    lin = lax.dot_general(conv, wl_ref[...], (((1,), (1,)), ((), ())),
                          preferred_element_type=jnp.float32) + bl_ref[...]
    mean = jnp.mean(lin, axis=1, keepdims=True)
    cent = lin - mean
    var = jnp.mean(cent * cent, axis=1, keepdims=True)
    o_ref[...] = cent * lax.rsqrt(var + EPS) * g_ref[...] + b_ref[...]


def _epilogue(partials, conv_bias, W_lin, b_lin, ln_gamma, ln_beta):
    BN = 1000
    return pl.pallas_call(
        _epi_body,
        grid=(N // BN,),
        in_specs=[
            pl.BlockSpec((2, BN, C), lambda i: (0, i, 0)),
            pl.BlockSpec((1, C), lambda i: (0, 0)),
            pl.BlockSpec((C, C), lambda i: (0, 0)),
            pl.BlockSpec((1, C), lambda i: (0, 0)),
            pl.BlockSpec((1, C), lambda i: (0, 0)),
            pl.BlockSpec((1, C), lambda i: (0, 0)),
        ],
        out_specs=pl.BlockSpec((BN, C), lambda i: (i, 0)),
        out_shape=jax.ShapeDtypeStruct((N, C), jnp.float32),
    )(partials, conv_bias.reshape(1, C), W_lin, b_lin.reshape(1, C),
      ln_gamma.reshape(1, C), ln_beta.reshape(1, C))


def kernel(feats, edge_index, edge_kernel, W_conv, conv_bias, W_lin, b_lin,
           ln_gamma, ln_beta):
    T = _transform(feats, W_conv).reshape(K * N, C)
    zeros = jnp.zeros((ROWS_PER_TILE, C), dtype=jnp.float32)
    partials = _sc_scatter(edge_index, edge_kernel, T, zeros)
    return _epilogue(partials, conv_bias, W_lin, b_lin, ln_gamma, ln_beta)
